# Initial kernel scaffold; baseline (speedup 1.0000x reference)
import jax
import jax.numpy as jnp
from jax.experimental import pallas as pl


def kernel(moe_inp, input_weight, output_weight, gate_w1, gate_b1, gate_w2):
    raise NotImplementedError("write your pallas kernel here")


# speedup vs baseline: 4.9811x; 4.9811x over previous
import functools

import jax
import jax.numpy as jnp
from jax import lax
from jax.experimental import pallas as pl
from jax.experimental.pallas import tpu as pltpu
from jax.experimental.pallas import tpu_sc as plsc

D = 1024
H = 2048
E = 16
K = 2
GS = 256
N = 4096
R = N * K
TM = 512
NT = R // TM
G = NT + E - 1
NC, NS = 2, 16
NW = NC * NS
CHUNK = 64


def _gating_body(x_ref, w1_ref, b1_ref, w2_ref, gates_ref, idx_ref, loss_ref):
    x = x_ref[...]
    hg = jax.nn.gelu(jnp.dot(x, w1_ref[...], preferred_element_type=jnp.float32)
                     + b1_ref[...])
    logits = jnp.dot(hg, w2_ref[...], preferred_element_type=jnp.float32)
    m = jnp.max(logits, axis=-1, keepdims=True)
    ex = jnp.exp(logits - m)
    probs = ex / jnp.sum(ex, axis=-1, keepdims=True)

    cols = lax.broadcasted_iota(jnp.int32, probs.shape, 1)
    m1 = jnp.max(probs, axis=-1, keepdims=True)
    i1 = jnp.min(jnp.where(probs == m1, cols, E), axis=-1, keepdims=True)
    masked = jnp.where(cols == i1, -jnp.inf, probs)
    m2 = jnp.max(masked, axis=-1, keepdims=True)
    i2 = jnp.min(jnp.where(masked == m2, cols, E), axis=-1, keepdims=True)
    denom = m1 + m2 + 1e-6
    gates_ref[...] = jnp.concatenate([m1 / denom, m2 / denom], axis=1)
    idx_ref[...] = jnp.concatenate([i1, i2], axis=1)

    eps = 1e-9
    pm = jnp.mean(probs, axis=0)
    h_marg = -jnp.sum(pm * jnp.log(pm + eps))
    h_cond = -jnp.mean(jnp.sum(probs * jnp.log(probs + eps), axis=-1))
    loss_ref[...] = jnp.reshape(h_cond - h_marg, (1, 1))


def _gating(x, gw1, gb1, gw2):
    return pl.pallas_call(
        _gating_body,
        out_shape=(
            jax.ShapeDtypeStruct((N, K), jnp.float32),
            jax.ShapeDtypeStruct((N, K), jnp.int32),
            jax.ShapeDtypeStruct((1, 1), jnp.float32),
        ),
    )(x, gw1, gb1.reshape(1, GS), gw2)


def _sc_gather_kernel(nrows_table):
    mesh = plsc.VectorSubcoreMesh(core_axis_name="c", subcore_axis_name="s")
    per_w = R // NW

    @functools.partial(
        pl.kernel,
        out_type=jax.ShapeDtypeStruct((R, D), jnp.float32),
        mesh=mesh,
        scratch_types=[
            pltpu.VMEM((CHUNK,), jnp.int32),
            pltpu.VMEM((CHUNK, D), jnp.float32),
            pltpu.SemaphoreType.DMA,
        ],
    )
    def gather_k(table_hbm, idx_hbm, out_hbm, idx_v, rows_v, sem):
        wid = lax.axis_index("s") * NC + lax.axis_index("c")
        for j in range(per_w // CHUNK):
            base = wid * per_w + j * CHUNK
            pltpu.sync_copy(idx_hbm.at[pl.ds(base, CHUNK)], idx_v)
            pltpu.async_copy(table_hbm.at[idx_v], rows_v, sem).wait()
            pltpu.sync_copy(rows_v, out_hbm.at[pl.ds(base, CHUNK)])

    return gather_k


def _gather_rows(table, idx):
    return _sc_gather_kernel(table.shape[0])(table, idx)


def _gmm_body(m_ids, e_ids, rs_ref, re_ref, xs_ref, win_ref, wout_ref, out_ref):
    w = pl.program_id(0)
    start = rs_ref[w]
    end = re_ref[w]
    m = m_ids[w]
    prev_m = m_ids[jnp.maximum(w - 1, 0)]
    is_first = jnp.logical_or(w == 0, m != prev_m)

    @pl.when(is_first)
    def _zero():
        out_ref[...] = jnp.zeros_like(out_ref)

    @pl.when(start < end)
    def _compute():
        x = xs_ref[...]
        h = jnp.maximum(
            jnp.dot(x, win_ref[0], preferred_element_type=jnp.float32), 0.0)
        o = jnp.dot(h, wout_ref[0], preferred_element_type=jnp.float32)
        rows = lax.broadcasted_iota(jnp.int32, (TM, 1), 0) + m * TM
        mask = jnp.logical_and(rows >= start, rows < end)
        out_ref[...] += jnp.where(mask, o, 0.0)


def _gmm(xs, w_in, w_out, m_ids, e_ids, rs, re):
    grid_spec = pltpu.PrefetchScalarGridSpec(
        num_scalar_prefetch=4,
        grid=(G,),
        in_specs=[
            pl.BlockSpec((TM, D), lambda w, m, e, s, t: (m[w], 0)),
            pl.BlockSpec((1, D, H), lambda w, m, e, s, t: (e[w], 0, 0)),
            pl.BlockSpec((1, H, D), lambda w, m, e, s, t: (e[w], 0, 0)),
        ],
        out_specs=pl.BlockSpec((TM, D), lambda w, m, e, s, t: (m[w], 0)),
    )
    return pl.pallas_call(
        _gmm_body,
        grid_spec=grid_spec,
        out_shape=jax.ShapeDtypeStruct((R, D), jnp.float32),
    )(m_ids, e_ids, rs, re, xs, w_in, w_out)


def _combine_body(g_ref, a_ref, b_ref, out_ref):
    g = g_ref[...]
    out_ref[...] = a_ref[...] * g[:, 0:1] + b_ref[...] * g[:, 1:2]


def _combine(gates, rc):
    nb = N // 512
    return pl.pallas_call(
        _combine_body,
        grid=(nb,),
        in_specs=[
            pl.BlockSpec((512, K), lambda i: (i, 0)),
            pl.BlockSpec((512, D), lambda i: (i, 0)),
            pl.BlockSpec((512, D), lambda i: (i + nb, 0)),
        ],
        out_specs=pl.BlockSpec((512, D), lambda i: (i, 0)),
        out_shape=jax.ShapeDtypeStruct((N, D), jnp.float32),
    )(gates, rc, rc)


def kernel(moe_inp, input_weight, output_weight, gate_w1, gate_b1, gate_w2):
    x = moe_inp.reshape(N, D)
    gates, idx, loss = _gating(x, gate_w1, gate_b1, gate_w2)

    flat_experts = idx.reshape(-1)
    sort_idx = jnp.argsort(flat_experts, stable=True).astype(jnp.int32)
    batch_index = sort_idx // K

    counts = jnp.bincount(flat_experts, length=E).astype(jnp.int32)
    ends = jnp.cumsum(counts)
    starts = ends - counts
    first_tile = starts // TM
    last_tile = jnp.where(counts > 0, (ends - 1) // TM, 0)
    ntiles = jnp.where(counts > 0, last_tile - first_tile + 1, 0)
    cum_excl = jnp.cumsum(ntiles) - ntiles
    w_act = cum_excl[-1] + ntiles[-1]

    ws = jnp.arange(G, dtype=jnp.int32)
    wc = jnp.minimum(ws, w_act - 1)
    e_w = (jnp.searchsorted(cum_excl, wc, side="right") - 1).astype(jnp.int32)
    m_w = (first_tile[e_w] + (wc - cum_excl[e_w])).astype(jnp.int32)
    rs_w = jnp.maximum(starts[e_w], m_w * TM)
    re_w = jnp.minimum(ends[e_w], (m_w + 1) * TM)
    valid = ws < w_act
    rs_w = jnp.where(valid, rs_w, 0).astype(jnp.int32)
    re_w = jnp.where(valid, re_w, 0).astype(jnp.int32)

    xs = _gather_rows(x, batch_index)
    os = _gmm(xs, input_weight, output_weight, m_w, e_w, rs_w, re_w)

    inv = jnp.zeros((R,), jnp.int32).at[sort_idx].set(
        jnp.arange(R, dtype=jnp.int32))
    invc = jnp.concatenate([inv[0::K], inv[1::K]])
    rc = _gather_rows(os, invc)

    out = _combine(gates, rc)
    return out.reshape(moe_inp.shape), loss[0, 0]
